# Initial kernel scaffold; baseline (speedup 1.0000x reference)
import functools

import jax
import jax.numpy as jnp
from jax import lax
from jax.experimental import pallas as pl
from jax.experimental.pallas import tpu as pltpu
from jax.experimental.pallas import tpu_sc as plsc

N = 10000
E = 320000
D = 128
NC, NS = 2, 16
NW = NC * NS
EPW = E // NW
K = 80
CHUNKS = EPW // K
SPAD = 10240
RPT = SPAD // NS
ZR = 128
DEGW = 16

_F32 = jnp.float32


def _make_sc_agg(with_deg: bool):
    mesh = plsc.VectorSubcoreMesh(core_axis_name="c", subcore_axis_name="s")
    out_type = [jax.ShapeDtypeStruct((NC, N, D), _F32)]
    scratch = [
        pltpu.VMEM((K,), jnp.int32),
        pltpu.VMEM((K,), jnp.int32),
        pltpu.VMEM((K, D), _F32),
        pltpu.VMEM((ZR, D), _F32),
        pltpu.VMEM_SHARED((SPAD, D), _F32),
    ]
    if with_deg:
        out_type.append(jax.ShapeDtypeStruct((NC, N, DEGW), _F32))
        scratch += [
            pltpu.VMEM((K, DEGW), _F32),
            pltpu.VMEM((RPT, DEGW), _F32),
            pltpu.VMEM_SHARED((SPAD, DEGW), _F32),
        ]

    def body(h_hbm, src_hbm, dst_hbm, *refs):
        if with_deg:
            (out_hbm, deg_hbm, src_v, dst_v, rows_v, z_v, acc,
             ones_v, dz_v, dacc) = refs
        else:
            out_hbm, src_v, dst_v, rows_v, z_v, acc = refs
        cid = lax.axis_index("c")
        sid = lax.axis_index("s")
        wid = cid * NS + sid

        @pl.loop(0, ZR)
        def _(r):
            @pl.loop(0, D, step=16)
            def _(c2):
                z_v[r, pl.ds(c2, 16)] = jnp.zeros((16,), _F32)

        if with_deg:
            @pl.loop(0, K)
            def _(r):
                ones_v[r, :] = jnp.ones((DEGW,), _F32)

            @pl.loop(0, RPT)
            def _(r):
                dz_v[r, :] = jnp.zeros((DEGW,), _F32)

        @pl.loop(0, RPT, step=ZR)
        def _(r0):
            pltpu.sync_copy(z_v, acc.at[pl.ds(sid * RPT + r0, ZR)])

        if with_deg:
            pltpu.sync_copy(dz_v, dacc.at[pl.ds(sid * RPT, RPT)])

        plsc.subcore_barrier()

        @pl.loop(0, CHUNKS)
        def _(ci):
            off = wid * EPW + ci * K
            pltpu.sync_copy(src_hbm.at[pl.ds(off, K)], src_v)
            pltpu.sync_copy(dst_hbm.at[pl.ds(off, K)], dst_v)
            pltpu.sync_copy(h_hbm.at[src_v], rows_v)
            pltpu.sync_copy(rows_v, acc.at[dst_v], add=True)
            if with_deg:
                pltpu.sync_copy(ones_v, dacc.at[dst_v], add=True)

        plsc.subcore_barrier()

        @pl.when(sid == 0)
        def _():
            pltpu.sync_copy(acc.at[pl.ds(0, N)], out_hbm.at[cid])
            if with_deg:
                pltpu.sync_copy(dacc.at[pl.ds(0, N)], deg_hbm.at[cid])

    return pl.kernel(body, out_type=tuple(out_type), mesh=mesh,
                     scratch_types=scratch)


_sc_agg_deg = _make_sc_agg(with_deg=True)
_sc_agg = _make_sc_agg(with_deg=False)

BR = 2000


def _dot(a, b):
    return jnp.dot(a, b, preferred_element_type=_F32,
                   precision=lax.Precision.HIGHEST)


def _tc_layer0(x, W0, b0, W1):
    def body(x_ref, w0_ref, b0_ref, w1_ref, o_ref):
        h = jnp.maximum(_dot(x_ref[...], w0_ref[...]) + b0_ref[...], 0.0)
        o_ref[...] = _dot(h, w1_ref[...])

    return pl.pallas_call(
        body,
        grid=(N // BR,),
        in_specs=[pl.BlockSpec((BR, D), lambda i: (i, 0)),
                  pl.BlockSpec((D, D), lambda i: (0, 0)),
                  pl.BlockSpec((1, D), lambda i: (0, 0)),
                  pl.BlockSpec((D, D), lambda i: (0, 0))],
        out_specs=pl.BlockSpec((BR, D), lambda i: (i, 0)),
        out_shape=jax.ShapeDtypeStruct((N, D), _F32),
    )(x, W0, b0.reshape(1, D), W1)


def _tc_combine(p, pdeg, b, W, b_out):
    DO = W.shape[1]

    def body(p_ref, pd_ref, b_ref, w_ref, bo_ref, o_ref):
        agg = p_ref[0] + p_ref[1]
        deg = pd_ref[0, :, 0:1] + pd_ref[1, :, 0:1]
        deg_inv = 1.0 / jnp.maximum(deg, 1.0)
        h = jnp.maximum(agg * deg_inv + b_ref[...], 0.0)
        o_ref[...] = _dot(h, w_ref[...]) + bo_ref[...]

    return pl.pallas_call(
        body,
        grid=(N // BR,),
        in_specs=[pl.BlockSpec((NC, BR, D), lambda i: (0, i, 0)),
                  pl.BlockSpec((NC, BR, DEGW), lambda i: (0, i, 0)),
                  pl.BlockSpec((1, D), lambda i: (0, 0)),
                  pl.BlockSpec((D, DO), lambda i: (0, 0)),
                  pl.BlockSpec((1, DO), lambda i: (0, 0))],
        out_specs=pl.BlockSpec((BR, DO), lambda i: (i, 0)),
        out_shape=jax.ShapeDtypeStruct((N, DO), _F32),
    )(p, pdeg, b.reshape(1, D), W, b_out.reshape(1, DO))


def kernel(x, edge_index, W0, b0, W1, b1, W2, b2, W3, b3, W4, b4, Wp, bp):
    ei = edge_index.astype(jnp.int32)
    src, dst = ei[0], ei[1]
    zero_b = jnp.zeros((D,), _F32)
    Wp_pad = jnp.concatenate([Wp, jnp.zeros((D, D - Wp.shape[1]), _F32)],
                             axis=1)
    bp_pad = jnp.concatenate([bp, jnp.zeros((D - bp.shape[0],), _F32)])

    h_lin = _tc_layer0(x, W0, b0, W1)
    p, pdeg = _sc_agg_deg(h_lin, src, dst)
    h_lin = _tc_combine(p, pdeg, b1, W2, zero_b)
    (p,) = _sc_agg(h_lin, src, dst)
    h_lin = _tc_combine(p, pdeg, b2, W3, zero_b)
    (p,) = _sc_agg(h_lin, src, dst)
    h_lin = _tc_combine(p, pdeg, b3, W4, zero_b)
    (p,) = _sc_agg(h_lin, src, dst)
    out = _tc_combine(p, pdeg, b4, Wp_pad, bp_pad)
    return out[:, :1]


# speedup vs baseline: 5.1318x; 5.1318x over previous
import functools

import jax
import jax.numpy as jnp
from jax import lax
from jax.experimental import pallas as pl
from jax.experimental.pallas import tpu as pltpu
from jax.experimental.pallas import tpu_sc as plsc

N = 10000
E = 320000
D = 128
NC, NS = 2, 16
NW = NC * NS
EPW = E // NW
K = 80
CHUNKS = EPW // K
SPAD = 10240
RPT = SPAD // NS
ZR = 128
DEGW = 16

_F32 = jnp.float32


def _make_sc_agg(with_deg: bool):
    mesh = plsc.VectorSubcoreMesh(core_axis_name="c", subcore_axis_name="s")
    out_type = [jax.ShapeDtypeStruct((NC, N, D), _F32)]
    scratch = [
        pltpu.VMEM((K,), jnp.int32),
        pltpu.VMEM((K,), jnp.int32),
        pltpu.VMEM((K, D), _F32),
        pltpu.VMEM((ZR, D), _F32),
        pltpu.VMEM_SHARED((SPAD, D), _F32),
    ]
    if with_deg:
        out_type.append(jax.ShapeDtypeStruct((NC, N, DEGW), _F32))
        scratch += [
            pltpu.VMEM((K, DEGW), _F32),
            pltpu.VMEM((RPT, DEGW), _F32),
            pltpu.VMEM_SHARED((SPAD, DEGW), _F32),
        ]

    def body(h_hbm, src_hbm, dst_hbm, *refs):
        if with_deg:
            (out_hbm, deg_hbm, src_v, dst_v, rows_v, z_v, acc,
             ones_v, dz_v, dacc) = refs
        else:
            out_hbm, src_v, dst_v, rows_v, z_v, acc = refs
        cid = lax.axis_index("c")
        sid = lax.axis_index("s")
        wid = cid * NS + sid

        @pl.loop(0, ZR)
        def _(r):
            @pl.loop(0, D, step=16)
            def _(c2):
                z_v[r, pl.ds(c2, 16)] = jnp.zeros((16,), _F32)

        if with_deg:
            @pl.loop(0, K)
            def _(r):
                ones_v[r, :] = jnp.ones((DEGW,), _F32)

            @pl.loop(0, RPT)
            def _(r):
                dz_v[r, :] = jnp.zeros((DEGW,), _F32)

        @pl.loop(0, RPT, step=ZR)
        def _(r0):
            pltpu.sync_copy(z_v, acc.at[pl.ds(sid * RPT + r0, ZR)])

        if with_deg:
            pltpu.sync_copy(dz_v, dacc.at[pl.ds(sid * RPT, RPT)])

        plsc.subcore_barrier()

        @pl.loop(0, CHUNKS)
        def _(ci):
            off = wid * EPW + ci * K
            pltpu.sync_copy(src_hbm.at[pl.ds(off, K)], src_v)
            pltpu.sync_copy(dst_hbm.at[pl.ds(off, K)], dst_v)
            pltpu.sync_copy(h_hbm.at[src_v], rows_v)
            pltpu.sync_copy(rows_v, acc.at[dst_v], add=True)
            if with_deg:
                pltpu.sync_copy(ones_v, dacc.at[dst_v], add=True)

        plsc.subcore_barrier()

        @pl.when(sid == 0)
        def _():
            pltpu.sync_copy(acc.at[pl.ds(0, N)], out_hbm.at[cid])
            if with_deg:
                pltpu.sync_copy(dacc.at[pl.ds(0, N)], deg_hbm.at[cid])

    return pl.kernel(body, out_type=tuple(out_type), mesh=mesh,
                     scratch_types=scratch,
                     compiler_params=pltpu.CompilerParams(
                         use_tc_tiling_on_sc=False))


_sc_agg_deg = _make_sc_agg(with_deg=True)
_sc_agg = _make_sc_agg(with_deg=False)

BR = 2000


def _dot(a, b):
    return jnp.dot(a, b, preferred_element_type=_F32,
                   precision=lax.Precision.HIGHEST)


def _tc_layer0(x, W0, b0, W1):
    def body(x_ref, w0_ref, b0_ref, w1_ref, o_ref):
        h = jnp.maximum(_dot(x_ref[...], w0_ref[...]) + b0_ref[...], 0.0)
        o_ref[...] = _dot(h, w1_ref[...])

    return pl.pallas_call(
        body,
        grid=(N // BR,),
        in_specs=[pl.BlockSpec((BR, D), lambda i: (i, 0)),
                  pl.BlockSpec((D, D), lambda i: (0, 0)),
                  pl.BlockSpec((1, D), lambda i: (0, 0)),
                  pl.BlockSpec((D, D), lambda i: (0, 0))],
        out_specs=pl.BlockSpec((BR, D), lambda i: (i, 0)),
        out_shape=jax.ShapeDtypeStruct((N, D), _F32),
    )(x, W0, b0.reshape(1, D), W1)


def _tc_combine(p, pdeg, b, W, b_out):
    DO = W.shape[1]

    def body(p_ref, pd_ref, b_ref, w_ref, bo_ref, o_ref):
        agg = p_ref[0] + p_ref[1]
        deg = pd_ref[0, :, 0:1] + pd_ref[1, :, 0:1]
        deg_inv = 1.0 / jnp.maximum(deg, 1.0)
        h = jnp.maximum(agg * deg_inv + b_ref[...], 0.0)
        o_ref[...] = _dot(h, w_ref[...]) + bo_ref[...]

    return pl.pallas_call(
        body,
        grid=(N // BR,),
        in_specs=[pl.BlockSpec((NC, BR, D), lambda i: (0, i, 0)),
                  pl.BlockSpec((NC, BR, DEGW), lambda i: (0, i, 0)),
                  pl.BlockSpec((1, D), lambda i: (0, 0)),
                  pl.BlockSpec((D, DO), lambda i: (0, 0)),
                  pl.BlockSpec((1, DO), lambda i: (0, 0))],
        out_specs=pl.BlockSpec((BR, DO), lambda i: (i, 0)),
        out_shape=jax.ShapeDtypeStruct((N, DO), _F32),
    )(p, pdeg, b.reshape(1, D), W, b_out.reshape(1, DO))


def kernel(x, edge_index, W0, b0, W1, b1, W2, b2, W3, b3, W4, b4, Wp, bp):
    ei = edge_index.astype(jnp.int32)
    src, dst = ei[0], ei[1]
    zero_b = jnp.zeros((D,), _F32)
    Wp_pad = jnp.concatenate([Wp, jnp.zeros((D, D - Wp.shape[1]), _F32)],
                             axis=1)
    bp_pad = jnp.concatenate([bp, jnp.zeros((D - bp.shape[0],), _F32)])

    h_lin = _tc_layer0(x, W0, b0, W1)
    p, pdeg = _sc_agg_deg(h_lin, src, dst)
    h_lin = _tc_combine(p, pdeg, b1, W2, zero_b)
    (p,) = _sc_agg(h_lin, src, dst)
    h_lin = _tc_combine(p, pdeg, b2, W3, zero_b)
    (p,) = _sc_agg(h_lin, src, dst)
    h_lin = _tc_combine(p, pdeg, b3, W4, zero_b)
    (p,) = _sc_agg(h_lin, src, dst)
    out = _tc_combine(p, pdeg, b4, Wp_pad, bp_pad)
    return out[:, :1]


# speedup vs baseline: 12.2459x; 2.3863x over previous
import jax
import jax.numpy as jnp
from jax import lax
from jax.experimental import pallas as pl
from jax.experimental.pallas import tpu as pltpu
from jax.experimental.pallas import tpu_sc as plsc

N = 10000
E = 320000
D = 128
NC, NS = 2, 16
NW = NC * NS
EPW = E // NW
K = 80
CHUNKS = EPW // K
SPAD = 10240
RPT = SPAD // NS
ZR = 32
DEGW = 16

_F32 = jnp.float32


def _make_sc_agg():
    mesh = plsc.VectorSubcoreMesh(core_axis_name="c", subcore_axis_name="s")
    scratch = [
        pltpu.VMEM((CHUNKS, K), jnp.int32),
        pltpu.VMEM((CHUNKS, K), jnp.int32),
        pltpu.VMEM((K, D), _F32),
        pltpu.VMEM((K, D), _F32),
        pltpu.VMEM((ZR, D), _F32),
        pltpu.VMEM_SHARED((SPAD, D), _F32),
        pltpu.SemaphoreType.DMA,
        pltpu.SemaphoreType.DMA,
    ]

    def body(h_hbm, src_hbm, dst_hbm, out_hbm, src_v, dst_v, rows0, rows1,
             z_v, acc, sem0, sem1):
        cid = lax.axis_index("c")
        sid = lax.axis_index("s")
        wid = cid * NS + sid

        pltpu.async_copy(src_hbm.at[pl.ds(wid * CHUNKS, CHUNKS)], src_v,
                         sem0)
        pltpu.async_copy(dst_hbm.at[pl.ds(wid * CHUNKS, CHUNKS)], dst_v,
                         sem1)

        @pl.loop(0, ZR)
        def _(r):
            @pl.loop(0, D, step=16)
            def _(c2):
                z_v[r, pl.ds(c2, 16)] = jnp.zeros((16,), _F32)

        pltpu.make_async_copy(src_hbm.at[pl.ds(wid * CHUNKS, CHUNKS)],
                              src_v, sem0).wait()
        pltpu.make_async_copy(dst_hbm.at[pl.ds(wid * CHUNKS, CHUNKS)],
                              dst_v, sem1).wait()

        @pl.loop(0, RPT, step=ZR)
        def _(r0):
            pltpu.sync_copy(z_v, acc.at[pl.ds(sid * RPT + r0, ZR)])

        plsc.subcore_barrier()

        def gather(ci, buf, sem):
            pltpu.async_copy(h_hbm.at[src_v.at[ci]], buf, sem)

        def wait_gather(ci, buf, sem):
            pltpu.make_async_copy(h_hbm.at[src_v.at[ci]], buf, sem).wait()

        gather(0, rows0, sem0)
        gather(1, rows1, sem1)

        @pl.loop(0, CHUNKS - 1, step=2)
        def _(ci):
            wait_gather(ci, rows0, sem0)
            pltpu.sync_copy(rows0, acc.at[dst_v.at[ci]], add=True)

            @pl.when(ci + 2 < CHUNKS)
            def _():
                gather(ci + 2, rows0, sem0)

            wait_gather(ci + 1, rows1, sem1)
            pltpu.sync_copy(rows1, acc.at[dst_v.at[ci + 1]], add=True)

            @pl.when(ci + 3 < CHUNKS)
            def _():
                gather(ci + 3, rows1, sem1)

        wait_gather(CHUNKS - 1, rows0, sem0)
        pltpu.sync_copy(rows0, acc.at[dst_v.at[CHUNKS - 1]], add=True)

        plsc.subcore_barrier()

        @pl.when(sid == 0)
        def _():
            pltpu.sync_copy(acc.at[pl.ds(0, N)], out_hbm.at[cid])

    return pl.kernel(body,
                     out_type=jax.ShapeDtypeStruct((NC, N, D), _F32),
                     mesh=mesh, scratch_types=scratch,
                     compiler_params=pltpu.CompilerParams(
                         use_tc_tiling_on_sc=False))


def _make_sc_deg():
    mesh = plsc.VectorSubcoreMesh(core_axis_name="c", subcore_axis_name="s")
    scratch = [
        pltpu.VMEM((CHUNKS, K), jnp.int32),
        pltpu.VMEM((K, DEGW), _F32),
        pltpu.VMEM((RPT, DEGW), _F32),
        pltpu.VMEM_SHARED((SPAD, DEGW), _F32),
        pltpu.SemaphoreType.DMA,
    ]

    def body(dst_hbm, deg_hbm, dst_v, ones_v, dz_v, dacc, sem0):
        cid = lax.axis_index("c")
        sid = lax.axis_index("s")
        wid = cid * NS + sid

        pltpu.async_copy(dst_hbm.at[pl.ds(wid * CHUNKS, CHUNKS)], dst_v,
                         sem0)

        @pl.loop(0, K)
        def _(r):
            ones_v[r, :] = jnp.ones((DEGW,), _F32)

        @pl.loop(0, RPT)
        def _(r):
            dz_v[r, :] = jnp.zeros((DEGW,), _F32)

        pltpu.make_async_copy(dst_hbm.at[pl.ds(wid * CHUNKS, CHUNKS)],
                              dst_v, sem0).wait()
        pltpu.sync_copy(dz_v, dacc.at[pl.ds(sid * RPT, RPT)])
        plsc.subcore_barrier()

        @pl.loop(0, CHUNKS)
        def _(ci):
            pltpu.sync_copy(ones_v, dacc.at[dst_v.at[ci]], add=True)

        plsc.subcore_barrier()

        @pl.when(sid == 0)
        def _():
            pltpu.sync_copy(dacc.at[pl.ds(0, N)], deg_hbm.at[cid])

    return pl.kernel(body,
                     out_type=jax.ShapeDtypeStruct((NC, N, DEGW), _F32),
                     mesh=mesh, scratch_types=scratch,
                     compiler_params=pltpu.CompilerParams(
                         use_tc_tiling_on_sc=False))


_sc_agg = _make_sc_agg()
_sc_deg = _make_sc_deg()

BR = 2000


def _dot(a, b):
    return jnp.dot(a, b, preferred_element_type=_F32)


def _tc_layer0(x, W0, b0, W1):
    def body(x_ref, w0_ref, b0_ref, w1_ref, o_ref):
        h = jnp.maximum(_dot(x_ref[...], w0_ref[...]) + b0_ref[...], 0.0)
        o_ref[...] = _dot(h, w1_ref[...])

    return pl.pallas_call(
        body,
        grid=(N // BR,),
        in_specs=[pl.BlockSpec((BR, D), lambda i: (i, 0)),
                  pl.BlockSpec((D, D), lambda i: (0, 0)),
                  pl.BlockSpec((1, D), lambda i: (0, 0)),
                  pl.BlockSpec((D, D), lambda i: (0, 0))],
        out_specs=pl.BlockSpec((BR, D), lambda i: (i, 0)),
        out_shape=jax.ShapeDtypeStruct((N, D), _F32),
    )(x, W0, b0.reshape(1, D), W1)


def _tc_combine(p, pdeg, b, W, b_out):
    DO = W.shape[1]

    def body(p_ref, pd_ref, b_ref, w_ref, bo_ref, o_ref):
        agg = p_ref[0] + p_ref[1]
        deg = pd_ref[0, :, 0:1] + pd_ref[1, :, 0:1]
        deg_inv = 1.0 / jnp.maximum(deg, 1.0)
        h = jnp.maximum(agg * deg_inv + b_ref[...], 0.0)
        o_ref[...] = _dot(h, w_ref[...]) + bo_ref[...]

    return pl.pallas_call(
        body,
        grid=(N // BR,),
        in_specs=[pl.BlockSpec((NC, BR, D), lambda i: (0, i, 0)),
                  pl.BlockSpec((NC, BR, DEGW), lambda i: (0, i, 0)),
                  pl.BlockSpec((1, D), lambda i: (0, 0)),
                  pl.BlockSpec((D, DO), lambda i: (0, 0)),
                  pl.BlockSpec((1, DO), lambda i: (0, 0))],
        out_specs=pl.BlockSpec((BR, DO), lambda i: (i, 0)),
        out_shape=jax.ShapeDtypeStruct((N, DO), _F32),
    )(p, pdeg, b.reshape(1, D), W, b_out.reshape(1, DO))


def kernel(x, edge_index, W0, b0, W1, b1, W2, b2, W3, b3, W4, b4, Wp, bp):
    ei = edge_index.astype(jnp.int32)
    src, dst = ei[0].reshape(E // K, K), ei[1].reshape(E // K, K)
    zero_b = jnp.zeros((D,), _F32)
    Wp_pad = jnp.concatenate([Wp, jnp.zeros((D, D - Wp.shape[1]), _F32)],
                             axis=1)
    bp_pad = jnp.concatenate([bp, jnp.zeros((D - bp.shape[0],), _F32)])

    pdeg = _sc_deg(dst)
    h_lin = _tc_layer0(x, W0, b0, W1)
    h_lin, pdeg = lax.optimization_barrier((h_lin, pdeg))
    p = _sc_agg(h_lin, src, dst)
    h_lin = _tc_combine(p, pdeg, b1, W2, zero_b)
    p = _sc_agg(h_lin, src, dst)
    h_lin = _tc_combine(p, pdeg, b2, W3, zero_b)
    p = _sc_agg(h_lin, src, dst)
    h_lin = _tc_combine(p, pdeg, b3, W4, zero_b)
    p = _sc_agg(h_lin, src, dst)
    out = _tc_combine(p, pdeg, b4, Wp_pad, bp_pad)
    return out[:, :1]
